# baseline (speedup 1.0000x reference)
import math

import jax
import jax.numpy as jnp
import numpy as np
from jax.experimental import pallas as pl
from jax.experimental.pallas import tpu as pltpu

_NUM_HEADS = 8
_NEG = -1e30


def _attn_kernel(tcp_ref, p_ref, wq_ref, bq_ref, wkv_ref, wo_ref,
                 bo_ref, mask_ref, maskt_ref, out_ref, kv_s):
    C = wq_ref.shape[0]
    H = _NUM_HEADS
    dh = C // H
    G, N, _ = p_ref.shape
    b = pl.program_id(0)
    par = jax.lax.rem(b, 2)
    kv_bf = kv_s[1 - par]
    q_cols = jax.lax.dot_general(
        wq_ref[...], tcp_ref[:, 0, :].astype(jnp.bfloat16),
        (((0,), (1,)), ((), ())),
        preferred_element_type=jnp.float32)
    q_cols = q_cols + bq_ref[...]
    qm_all = jnp.concatenate(
        [q_cols[:, g:g + 1] * mask_ref[...] for g in range(G)],
        axis=1).astype(jnp.bfloat16)
    scores_all = jnp.dot(kv_bf[:, :C], qm_all,
                         preferred_element_type=jnp.float32) / np.float32(
                             math.sqrt(dh))
    scores = jnp.concatenate(
        [scores_all[g * N:(g + 1) * N, g * H:(g + 1) * H]
         for g in range(G)], axis=1)
    m = jnp.max(scores, axis=0, keepdims=True)
    e = jnp.exp(scores - m)
    attn = e / jnp.sum(e, axis=0, keepdims=True)
    rowblk = jax.lax.broadcasted_iota(jnp.int32, (G * N, G * H), 0) // N
    colblk = jax.lax.broadcasted_iota(jnp.int32, (G * N, G * H), 1) // H
    attn_full = jnp.where(
        rowblk == colblk,
        jnp.concatenate([attn] * G, axis=0), 0.0)
    ctx_all = jax.lax.dot_general(
        attn_full.astype(jnp.bfloat16), kv_bf[:, C:],
        (((0,), (0,)), ((), ())),
        preferred_element_type=jnp.float32)
    out_rows = [
        jnp.sum(ctx_all[g * H:(g + 1) * H] * maskt_ref[...], axis=0,
                keepdims=True) for g in range(G)]
    out_mat = jnp.concatenate(out_rows, axis=0)
    moe = jnp.dot(out_mat.astype(jnp.bfloat16), wo_ref[...],
                  preferred_element_type=jnp.float32) + bo_ref[...]
    out_ref[...] = moe[:, None, :]

    patches = p_ref[...].reshape(G * N, C).astype(jnp.bfloat16)
    kv = jnp.dot(patches, wkv_ref[...],
                 preferred_element_type=jnp.float32)
    kv_s[par] = kv.astype(jnp.bfloat16)


def _moe_kernel(min_ref, vis_ref, prm_ref, gw_ref, gb_ref, ew_ref, eb_ref,
                rw_ref, rb_ref, bnw_ref, bnb_ref, clsw_ref, clsb_ref,
                lab_ref, enh_ref, loss_ref):
    B, C = min_ref.shape
    num_attrs = prm_ref.shape[0]
    num_exp = ew_ref.shape[0]
    ncls_pad = clsw_ref.shape[2]

    base = min_ref[...]
    vis = vis_ref[...]

    iota_e = jax.lax.broadcasted_iota(jnp.int32, (1, num_exp), 1)
    iota_c = jax.lax.broadcasted_iota(jnp.int32, (1, ncls_pad), 1)
    loss = jnp.zeros((1, 1), jnp.float32)
    moe_outs = []
    score_cols = []
    for i in range(num_attrs):
        x = (base + prm_ref[i:i + 1, :]) + vis
        x_bf = x.astype(jnp.bfloat16)
        gate = jnp.dot(x_bf, gw_ref[...],
                       preferred_element_type=jnp.float32) + gb_ref[...]
        mn = jnp.min(gate, axis=1, keepdims=True)
        is_mn = gate == mn
        drop_idx = jnp.max(jnp.where(is_mn, iota_e, -1), axis=1,
                           keepdims=True)
        z = jnp.where(iota_e == drop_idx, _NEG, gate)
        zm = jnp.max(z, axis=1, keepdims=True)
        w = jnp.exp(z - zm)
        w = w / jnp.sum(w, axis=1, keepdims=True)
        mo = jnp.zeros((B, C), jnp.float32)
        for e_i in range(num_exp):
            eo = jnp.dot(x_bf, ew_ref[e_i],
                         preferred_element_type=jnp.float32) \
                + eb_ref[e_i:e_i + 1, :]
            mo = mo + w[:, e_i:e_i + 1] * eo
        moe_outs.append(mo)
        score = jnp.dot(x_bf, rw_ref[...],
                        preferred_element_type=jnp.float32) + rb_ref[...]
        score_cols.append(jnp.mean(score, axis=1, keepdims=True))

        mean = jnp.mean(mo, axis=0, keepdims=True)
        var = jnp.mean((mo - mean) * (mo - mean), axis=0, keepdims=True)
        feat = (mo - mean) / jnp.sqrt(var + 1e-5) * bnw_ref[i:i + 1, :] \
            + bnb_ref[i:i + 1, :]
        logits = jnp.dot(feat.astype(jnp.bfloat16), clsw_ref[i],
                         preferred_element_type=jnp.float32) \
            + clsb_ref[i:i + 1, :]
        lm = jnp.max(logits, axis=1, keepdims=True)
        lse = lm + jnp.log(jnp.sum(jnp.exp(logits - lm), axis=1,
                                   keepdims=True))
        oh = iota_c == lab_ref[:, i:i + 1]
        picked = jnp.sum(jnp.where(oh, logits, 0.0), axis=1, keepdims=True)
        loss = loss + jnp.sum(lse - picked, axis=0, keepdims=True) \
            * np.float32(1.0 / B)

    scores = jnp.concatenate(score_cols, axis=1)
    iota_a = jax.lax.broadcasted_iota(jnp.int32, (1, num_attrs), 1)
    k = int(num_attrs * 0.7)
    keep = jnp.ones(scores.shape, jnp.bool_)
    for _ in range(num_attrs - k):
        cur = jnp.where(keep, scores, jnp.float32(1e30))
        mn = jnp.min(cur, axis=1, keepdims=True)
        is_mn = jnp.logical_and(cur == mn, keep)
        drop_idx = jnp.max(jnp.where(is_mn, iota_a, -1), axis=1,
                           keepdims=True)
        keep = jnp.logical_and(keep, iota_a != drop_idx)
    z = jnp.where(keep, scores, _NEG)
    zm = jnp.max(z, axis=1, keepdims=True)
    wz = jnp.exp(z - zm)
    wz = wz / jnp.sum(wz, axis=1, keepdims=True)
    enh = jnp.zeros((B, C), jnp.float32)
    for i in range(num_attrs):
        enh = enh + wz[:, i:i + 1] * moe_outs[i]
    enh_ref[...] = enh
    loss_ref[...] = loss


def kernel(text_cls, visual_cls, visual_patchs, attr_labels, params):
    B, N, C = visual_patchs.shape
    H = _NUM_HEADS
    dh = C // H
    num_attrs = params["bn_w"].shape[0]
    bf = jnp.bfloat16

    headmask = jnp.asarray(
        (np.arange(C)[:, None] // dh) == np.arange(H)[None, :], jnp.float32)
    headmask_t = headmask.T

    G = 4
    nblk = B // G
    wkv = jnp.concatenate([params["ca_wk"], params["ca_wv"]],
                          axis=1).astype(bf)
    moe_in = pl.pallas_call(
        _attn_kernel,
        grid=(nblk + 1,),
        in_specs=[
            pl.BlockSpec((G, 1, C),
                         lambda b: (jnp.maximum(b - 1, 0), 0, 0)),
            pl.BlockSpec((G, N, C),
                         lambda b: (jnp.minimum(b, nblk - 1), 0, 0)),
            pl.BlockSpec((C, C), lambda b: (0, 0)),
            pl.BlockSpec((C, 1), lambda b: (0, 0)),
            pl.BlockSpec((C, 2 * C), lambda b: (0, 0)),
            pl.BlockSpec((C, C), lambda b: (0, 0)),
            pl.BlockSpec((1, C), lambda b: (0, 0)),
            pl.BlockSpec((C, H), lambda b: (0, 0)),
            pl.BlockSpec((H, C), lambda b: (0, 0)),
        ],
        out_specs=pl.BlockSpec((G, 1, C),
                               lambda b: (jnp.maximum(b - 1, 0), 0, 0)),
        out_shape=jax.ShapeDtypeStruct((B, 1, C), jnp.float32),
        scratch_shapes=[pltpu.VMEM((2, G * N, 2 * C), jnp.bfloat16)],
    )(text_cls, visual_patchs,
      params["ca_wq"].astype(bf), params["ca_bq"][:, None],
      wkv, params["ca_wo"].astype(bf), params["ca_bo"][None, :],
      headmask, headmask_t)
    moe_in = moe_in[:, 0, :]

    ncls_pad = 16
    clsw = jnp.stack([
        jnp.pad(w.T, ((0, 0), (0, ncls_pad - w.shape[0])))
        for w in params["cls_w"]]).astype(bf)
    clsb = jnp.asarray(np.stack([
        np.where(np.arange(ncls_pad) < w_nc, 0.0, _NEG)
        for w_nc in [w.shape[0] for w in params["cls_w"]]]), jnp.float32)

    num_exp = params["expert_w"].shape[0]
    enh, loss = pl.pallas_call(
        _moe_kernel,
        in_specs=[
            pl.BlockSpec((B, C), lambda: (0, 0)),
            pl.BlockSpec((B, C), lambda: (0, 0)),
            pl.BlockSpec((num_attrs, C), lambda: (0, 0)),
            pl.BlockSpec((C, num_exp), lambda: (0, 0)),
            pl.BlockSpec((1, num_exp), lambda: (0, 0)),
            pl.BlockSpec((num_exp, C, C), lambda: (0, 0, 0)),
            pl.BlockSpec((num_exp, C), lambda: (0, 0)),
            pl.BlockSpec((C, C), lambda: (0, 0)),
            pl.BlockSpec((1, C), lambda: (0, 0)),
            pl.BlockSpec((num_attrs, C), lambda: (0, 0)),
            pl.BlockSpec((num_attrs, C), lambda: (0, 0)),
            pl.BlockSpec((num_attrs, C, ncls_pad), lambda: (0, 0, 0)),
            pl.BlockSpec((num_attrs, ncls_pad), lambda: (0, 0)),
            pl.BlockSpec((B, num_attrs), lambda: (0, 0)),
        ],
        out_specs=[
            pl.BlockSpec((B, C), lambda: (0, 0)),
            pl.BlockSpec((1, 1), lambda: (0, 0)),
        ],
        out_shape=[
            jax.ShapeDtypeStruct((B, C), jnp.float32),
            jax.ShapeDtypeStruct((1, 1), jnp.float32),
        ],
    )(moe_in, visual_cls, params["prompt"][0],
      params["gate_w"].astype(bf), params["gate_b"][None, :],
      params["expert_w"].astype(bf), params["expert_b"],
      params["router_w"].astype(bf), params["router_b"][None, :],
      params["bn_w"], params["bn_b"], clsw, clsb,
      attr_labels.astype(jnp.int32))
    return enh, loss[0, 0]


# speedup vs baseline: 1.1213x; 1.1213x over previous
import math

import jax
import jax.numpy as jnp
import numpy as np
from jax.experimental import pallas as pl

_NUM_HEADS = 8
_NEG = -1e30


def _attn_kernel(tc_ref, p_ref, wq_ref, bq_ref, wkv_ref, wo_ref,
                 bo_ref, mask_ref, maskt_ref, out_ref):
    C = wq_ref.shape[0]
    H = _NUM_HEADS
    dh = C // H
    G, N, _ = p_ref.shape
    q_cols = jax.lax.dot_general(
        wq_ref[...], tc_ref[:, 0, :].astype(jnp.bfloat16),
        (((0,), (1,)), ((), ())),
        preferred_element_type=jnp.float32)
    q_cols = q_cols + bq_ref[...]
    patches = p_ref[...].reshape(G * N, C).astype(jnp.bfloat16)
    kv = jnp.dot(patches, wkv_ref[...],
                 preferred_element_type=jnp.float32)
    kv_bf = kv.astype(jnp.bfloat16)
    qm_all = jnp.concatenate(
        [q_cols[:, g:g + 1] * mask_ref[...] for g in range(G)],
        axis=1).astype(jnp.bfloat16)
    scores_all = jnp.dot(kv_bf[:, :C], qm_all,
                         preferred_element_type=jnp.float32) / np.float32(
                             math.sqrt(dh))
    scores = jnp.concatenate(
        [scores_all[g * N:(g + 1) * N, g * H:(g + 1) * H]
         for g in range(G)], axis=1)
    m = jnp.max(scores, axis=0, keepdims=True)
    e = jnp.exp(scores - m)
    attn = e / jnp.sum(e, axis=0, keepdims=True)
    rowblk = jax.lax.broadcasted_iota(jnp.int32, (G * N, G * H), 0) // N
    colblk = jax.lax.broadcasted_iota(jnp.int32, (G * N, G * H), 1) // H
    attn_full = jnp.where(
        rowblk == colblk,
        jnp.concatenate([attn] * G, axis=0), 0.0)
    ctx_all = jax.lax.dot_general(
        attn_full.astype(jnp.bfloat16), kv_bf[:, C:],
        (((0,), (0,)), ((), ())),
        preferred_element_type=jnp.float32)
    out_rows = [
        jnp.sum(ctx_all[g * H:(g + 1) * H] * maskt_ref[...], axis=0,
                keepdims=True) for g in range(G)]
    out_mat = jnp.concatenate(out_rows, axis=0)
    moe = jnp.dot(out_mat.astype(jnp.bfloat16), wo_ref[...],
                  preferred_element_type=jnp.float32) + bo_ref[...]
    out_ref[...] = moe[:, None, :]


def _moe_kernel(min_ref, vis_ref, prm_ref, gw_ref, gb_ref, ew_ref, eb_ref,
                rw_ref, rb_ref, bnw_ref, bnb_ref, clsw_ref, clsb_ref,
                lab_ref, enh_ref, loss_ref):
    B, C = min_ref.shape
    num_attrs = prm_ref.shape[0]
    num_exp = ew_ref.shape[0]
    ncls_pad = clsw_ref.shape[2]

    base = min_ref[...]
    vis = vis_ref[...]

    iota_e = jax.lax.broadcasted_iota(jnp.int32, (1, num_exp), 1)
    iota_c = jax.lax.broadcasted_iota(jnp.int32, (1, ncls_pad), 1)
    loss = jnp.zeros((1, 1), jnp.float32)
    X_bf = jnp.concatenate(
        [(base + prm_ref[i:i + 1, :]) + vis for i in range(num_attrs)],
        axis=0).astype(jnp.bfloat16)
    gate_all = jnp.dot(X_bf, gw_ref[...],
                       preferred_element_type=jnp.float32) + gb_ref[...]
    score_all = jnp.dot(X_bf, rw_ref[...],
                        preferred_element_type=jnp.float32) + rb_ref[...]
    eo_all = [jnp.dot(X_bf, ew_ref[e_i],
                      preferred_element_type=jnp.float32)
              for e_i in range(num_exp)]
    moe_outs = []
    score_cols = []
    for i in range(num_attrs):
        r0 = i * B
        gate = gate_all[r0:r0 + B]
        mn = jnp.min(gate, axis=1, keepdims=True)
        is_mn = gate == mn
        drop_idx = jnp.max(jnp.where(is_mn, iota_e, -1), axis=1,
                           keepdims=True)
        z = jnp.where(iota_e == drop_idx, _NEG, gate)
        zm = jnp.max(z, axis=1, keepdims=True)
        w = jnp.exp(z - zm)
        w = w / jnp.sum(w, axis=1, keepdims=True)
        mo = jnp.zeros((B, C), jnp.float32)
        for e_i in range(num_exp):
            eo = eo_all[e_i][r0:r0 + B] + eb_ref[e_i:e_i + 1, :]
            mo = mo + w[:, e_i:e_i + 1] * eo
        moe_outs.append(mo)
        score_cols.append(jnp.mean(score_all[r0:r0 + B], axis=1,
                                   keepdims=True))

        mean = jnp.mean(mo, axis=0, keepdims=True)
        var = jnp.mean((mo - mean) * (mo - mean), axis=0, keepdims=True)
        feat = (mo - mean) / jnp.sqrt(var + 1e-5) * bnw_ref[i:i + 1, :] \
            + bnb_ref[i:i + 1, :]
        logits = jnp.dot(feat.astype(jnp.bfloat16), clsw_ref[i],
                         preferred_element_type=jnp.float32) \
            + clsb_ref[i:i + 1, :]
        lm = jnp.max(logits, axis=1, keepdims=True)
        lse = lm + jnp.log(jnp.sum(jnp.exp(logits - lm), axis=1,
                                   keepdims=True))
        oh = iota_c == lab_ref[:, i:i + 1]
        picked = jnp.sum(jnp.where(oh, logits, 0.0), axis=1, keepdims=True)
        loss = loss + jnp.sum(lse - picked, axis=0, keepdims=True) \
            * np.float32(1.0 / B)

    scores = jnp.concatenate(score_cols, axis=1)
    iota_a = jax.lax.broadcasted_iota(jnp.int32, (1, num_attrs), 1)
    k = int(num_attrs * 0.7)
    keep = jnp.ones(scores.shape, jnp.bool_)
    for _ in range(num_attrs - k):
        cur = jnp.where(keep, scores, jnp.float32(1e30))
        mn = jnp.min(cur, axis=1, keepdims=True)
        is_mn = jnp.logical_and(cur == mn, keep)
        drop_idx = jnp.max(jnp.where(is_mn, iota_a, -1), axis=1,
                           keepdims=True)
        keep = jnp.logical_and(keep, iota_a != drop_idx)
    z = jnp.where(keep, scores, _NEG)
    zm = jnp.max(z, axis=1, keepdims=True)
    wz = jnp.exp(z - zm)
    wz = wz / jnp.sum(wz, axis=1, keepdims=True)
    enh = jnp.zeros((B, C), jnp.float32)
    for i in range(num_attrs):
        enh = enh + wz[:, i:i + 1] * moe_outs[i]
    enh_ref[...] = enh
    loss_ref[...] = loss


def kernel(text_cls, visual_cls, visual_patchs, attr_labels, params):
    B, N, C = visual_patchs.shape
    H = _NUM_HEADS
    dh = C // H
    num_attrs = params["bn_w"].shape[0]
    bf = jnp.bfloat16

    headmask = jnp.asarray(
        (np.arange(C)[:, None] // dh) == np.arange(H)[None, :], jnp.float32)
    headmask_t = headmask.T

    G = 4
    wkv = jnp.concatenate([params["ca_wk"], params["ca_wv"]],
                          axis=1).astype(bf)
    moe_in = pl.pallas_call(
        _attn_kernel,
        grid=(B // G,),
        in_specs=[
            pl.BlockSpec((G, 1, C), lambda b: (b, 0, 0)),
            pl.BlockSpec((G, N, C), lambda b: (b, 0, 0)),
            pl.BlockSpec((C, C), lambda b: (0, 0)),
            pl.BlockSpec((C, 1), lambda b: (0, 0)),
            pl.BlockSpec((C, 2 * C), lambda b: (0, 0)),
            pl.BlockSpec((C, C), lambda b: (0, 0)),
            pl.BlockSpec((1, C), lambda b: (0, 0)),
            pl.BlockSpec((C, H), lambda b: (0, 0)),
            pl.BlockSpec((H, C), lambda b: (0, 0)),
        ],
        out_specs=pl.BlockSpec((G, 1, C), lambda b: (b, 0, 0)),
        out_shape=jax.ShapeDtypeStruct((B, 1, C), jnp.float32),
    )(text_cls, visual_patchs,
      params["ca_wq"].astype(bf), params["ca_bq"][:, None],
      wkv, params["ca_wo"].astype(bf), params["ca_bo"][None, :],
      headmask, headmask_t)
    moe_in = moe_in[:, 0, :]

    ncls_pad = 16
    clsw = jnp.stack([
        jnp.pad(w.T, ((0, 0), (0, ncls_pad - w.shape[0])))
        for w in params["cls_w"]]).astype(bf)
    clsb = jnp.asarray(np.stack([
        np.where(np.arange(ncls_pad) < w_nc, 0.0, _NEG)
        for w_nc in [w.shape[0] for w in params["cls_w"]]]), jnp.float32)

    num_exp = params["expert_w"].shape[0]
    enh, loss = pl.pallas_call(
        _moe_kernel,
        in_specs=[
            pl.BlockSpec((B, C), lambda: (0, 0)),
            pl.BlockSpec((B, C), lambda: (0, 0)),
            pl.BlockSpec((num_attrs, C), lambda: (0, 0)),
            pl.BlockSpec((C, num_exp), lambda: (0, 0)),
            pl.BlockSpec((1, num_exp), lambda: (0, 0)),
            pl.BlockSpec((num_exp, C, C), lambda: (0, 0, 0)),
            pl.BlockSpec((num_exp, C), lambda: (0, 0)),
            pl.BlockSpec((C, C), lambda: (0, 0)),
            pl.BlockSpec((1, C), lambda: (0, 0)),
            pl.BlockSpec((num_attrs, C), lambda: (0, 0)),
            pl.BlockSpec((num_attrs, C), lambda: (0, 0)),
            pl.BlockSpec((num_attrs, C, ncls_pad), lambda: (0, 0, 0)),
            pl.BlockSpec((num_attrs, ncls_pad), lambda: (0, 0)),
            pl.BlockSpec((B, num_attrs), lambda: (0, 0)),
        ],
        out_specs=[
            pl.BlockSpec((B, C), lambda: (0, 0)),
            pl.BlockSpec((1, 1), lambda: (0, 0)),
        ],
        out_shape=[
            jax.ShapeDtypeStruct((B, C), jnp.float32),
            jax.ShapeDtypeStruct((1, 1), jnp.float32),
        ],
    )(moe_in, visual_cls, params["prompt"][0],
      params["gate_w"].astype(bf), params["gate_b"][None, :],
      params["expert_w"].astype(bf), params["expert_b"],
      params["router_w"].astype(bf), params["router_b"][None, :],
      params["bn_w"], params["bn_b"], clsw, clsb,
      attr_labels.astype(jnp.int32))
    return enh, loss[0, 0]
